# Initial kernel scaffold; baseline (speedup 1.0000x reference)
import jax
import jax.numpy as jnp
from jax.experimental import pallas as pl


def kernel(x, edge_index, W1, b1, W2, b2):
    raise NotImplementedError("write your pallas kernel here")


# speedup vs baseline: 4.1616x; 4.1616x over previous
import functools

import jax
import jax.numpy as jnp
from jax import lax
from jax.experimental import pallas as pl
from jax.experimental.pallas import tpu as pltpu
from jax.experimental.pallas import tpu_sc as plsc

NC = 2
NS = 16
B = 128
R = 1000


def _mesh():
    return plsc.VectorSubcoreMesh(
        core_axis_name="c", subcore_axis_name="s", num_cores=NC, num_subcores=NS
    )


def _rpt(N):
    return -(-N // (NS * 8)) * 8


def _make_spmm(N, CH, F):
    CHH = CH // NC
    RPT = _rpt(N)

    @functools.partial(
        pl.kernel,
        out_type=jax.ShapeDtypeStruct((NC, N, F), jnp.float32),
        mesh=_mesh(),
        scratch_types=[
            pltpu.VMEM((CHH, B), jnp.int32),
            pltpu.VMEM((CHH, B), jnp.int32),
            pltpu.VMEM((B, F), jnp.float32),
            pltpu.VMEM_SHARED((N + 8, F), jnp.float32),
        ],
    )
    def spmm_kernel(src_hbm, dst_hbm, g_hbm, zeros_hbm, out, src_v, dst_v, rows_v, acc):
        c = lax.axis_index("c")
        s = lax.axis_index("s")
        row0 = jnp.minimum(s * RPT, N - RPT)
        pltpu.sync_copy(src_hbm.at[s, pl.ds(c * CHH, CHH)], src_v)
        pltpu.sync_copy(dst_hbm.at[s, pl.ds(c * CHH, CHH)], dst_v)
        pltpu.sync_copy(
            zeros_hbm.at[pl.ds(row0, RPT)], acc.at[pl.ds(row0, RPT)]
        )
        plsc.subcore_barrier()

        def body(j, carry):
            pltpu.sync_copy(g_hbm.at[src_v.at[j]], rows_v)
            pltpu.sync_copy(rows_v, acc.at[dst_v.at[j]], add=True)
            return carry

        lax.fori_loop(0, CHH, body, 0)
        plsc.subcore_barrier()
        pltpu.sync_copy(
            acc.at[pl.ds(row0, RPT)], out.at[c, pl.ds(row0, RPT)]
        )

    return spmm_kernel


def _dinv_block(d0, d1):
    d = d0[:, 0:1] + d1[:, 0:1]
    return jnp.where(d > 0.0, lax.rsqrt(d), 0.0)


def _pre_body(x_ref, d0_ref, d1_ref, g_ref):
    dinv = _dinv_block(d0_ref[...], d1_ref[...])
    g_ref[...] = x_ref[...] * dinv


def _mid_body(s_ref, d0_ref, d1_ref, tx_ref, g_ref):
    dinv = _dinv_block(d0_ref[...], d1_ref[...])
    sfull = s_ref[0] + s_ref[1]
    tx = -(dinv * sfull)
    tx_ref[...] = tx
    g_ref[...] = dinv * tx


def _comb1_body(x_ref, tx1_ref, s2_ref, d0_ref, d1_ref, w_ref, b_ref, h_ref, g_ref):
    dinv = _dinv_block(d0_ref[...], d1_ref[...])
    s2 = s2_ref[0] + s2_ref[1]
    xb = x_ref[...]
    tx2 = -2.0 * (dinv * s2) - xb
    acc = (
        jnp.dot(xb, w_ref[0], preferred_element_type=jnp.float32)
        + jnp.dot(tx1_ref[...], w_ref[1], preferred_element_type=jnp.float32)
        + jnp.dot(tx2, w_ref[2], preferred_element_type=jnp.float32)
        + b_ref[...]
    )
    h = jnp.maximum(acc, 0.0)
    h_ref[...] = h
    g_ref[...] = dinv * h


def _comb2_body(h_ref, ty1_ref, s4_ref, d0_ref, d1_ref, w_ref, b_ref, o_ref):
    dinv = _dinv_block(d0_ref[...], d1_ref[...])
    s4 = s4_ref[0] + s4_ref[1]
    hb = h_ref[...]
    ty2 = -2.0 * (dinv * s4) - hb
    logits = (
        jnp.dot(hb, w_ref[0], preferred_element_type=jnp.float32)
        + jnp.dot(ty1_ref[...], w_ref[1], preferred_element_type=jnp.float32)
        + jnp.dot(ty2, w_ref[2], preferred_element_type=jnp.float32)
        + b_ref[...]
    )
    m = jnp.max(logits, axis=1, keepdims=True)
    lse = jnp.log(jnp.sum(jnp.exp(logits - m), axis=1, keepdims=True)) + m
    o_ref[...] = logits - lse


def _row_spec(w):
    return pl.BlockSpec((R, w), lambda i: (i, 0))


def _pair_spec(w):
    return pl.BlockSpec((2, R, w), lambda i: (0, i, 0))


def kernel(x, edge_index, W1, b1, W2, b2):
    N, F = x.shape
    E = edge_index.shape[1]
    H = W1.shape[2]
    C = W2.shape[2]
    NB = N // R

    CH = -(-E // (NS * B * 16)) * 16
    Epad = NS * CH * B
    src = edge_index[0]
    dst = edge_index[1]
    pad = Epad - E
    srcp = jnp.concatenate([src, jnp.zeros((pad,), jnp.int32)]).reshape(NS, CH, B)
    dstp = jnp.concatenate([dst, jnp.full((pad,), N, jnp.int32)]).reshape(NS, CH, B)
    zerosF = jnp.zeros((N, F), jnp.float32)
    onesF = jnp.ones((N, F), jnp.float32)

    spmm = _make_spmm(N, CH, F)
    degp = spmm(srcp, dstp, onesF, zerosF)
    deg0 = degp[0, :, :8]
    deg1 = degp[1, :, :8]

    dspec = _row_spec(8)

    pre = pl.pallas_call(
        _pre_body,
        grid=(NB,),
        in_specs=[_row_spec(F), dspec, dspec],
        out_specs=_row_spec(F),
        out_shape=jax.ShapeDtypeStruct((N, F), jnp.float32),
    )
    mid = pl.pallas_call(
        _mid_body,
        grid=(NB,),
        in_specs=[_pair_spec(F), dspec, dspec],
        out_specs=[_row_spec(F), _row_spec(F)],
        out_shape=[
            jax.ShapeDtypeStruct((N, F), jnp.float32),
            jax.ShapeDtypeStruct((N, F), jnp.float32),
        ],
    )
    comb1 = pl.pallas_call(
        _comb1_body,
        grid=(NB,),
        in_specs=[
            _row_spec(F),
            _row_spec(F),
            _pair_spec(F),
            dspec,
            dspec,
            pl.BlockSpec((3, F, H), lambda i: (0, 0, 0)),
            pl.BlockSpec((1, H), lambda i: (0, 0)),
        ],
        out_specs=[_row_spec(H), _row_spec(H)],
        out_shape=[
            jax.ShapeDtypeStruct((N, H), jnp.float32),
            jax.ShapeDtypeStruct((N, H), jnp.float32),
        ],
    )
    comb2 = pl.pallas_call(
        _comb2_body,
        grid=(NB,),
        in_specs=[
            _row_spec(H),
            _row_spec(H),
            _pair_spec(H),
            dspec,
            dspec,
            pl.BlockSpec((3, H, C), lambda i: (0, 0, 0)),
            pl.BlockSpec((1, C), lambda i: (0, 0)),
        ],
        out_specs=_row_spec(C),
        out_shape=jax.ShapeDtypeStruct((N, C), jnp.float32),
    )

    g0 = pre(x, deg0, deg1)
    s1 = spmm(srcp, dstp, g0, zerosF)
    tx1, g1 = mid(s1, deg0, deg1)
    s2 = spmm(srcp, dstp, g1, zerosF)
    h, g2 = comb1(x, tx1, s2, deg0, deg1, W1, b1.reshape(1, H))
    s3 = spmm(srcp, dstp, g2, zerosF)
    ty1, g3 = mid(s3, deg0, deg1)
    s4 = spmm(srcp, dstp, g3, zerosF)
    out = comb2(h, ty1, s4, deg0, deg1, W2, b2.reshape(1, C))
    return out


# speedup vs baseline: 4.7420x; 1.1395x over previous
import functools

import jax
import jax.numpy as jnp
from jax import lax
from jax.experimental import pallas as pl
from jax.experimental.pallas import tpu as pltpu
from jax.experimental.pallas import tpu_sc as plsc

NC = 2
NS = 16
B = 128
R = 1000


def _mesh():
    return plsc.VectorSubcoreMesh(
        core_axis_name="c", subcore_axis_name="s", num_cores=NC, num_subcores=NS
    )


def _rpt(N):
    return -(-N // (NS * 8)) * 8


def _make_spmm(N, CH, F):
    CHH = CH // NC
    RPT = _rpt(N)

    @functools.partial(
        pl.kernel,
        out_type=jax.ShapeDtypeStruct((NC, N, F), jnp.float32),
        mesh=_mesh(),
        scratch_types=[
            pltpu.VMEM((16, B), jnp.int32),
            pltpu.VMEM((16, B), jnp.int32),
            pltpu.VMEM((B, F), jnp.float32),
            pltpu.VMEM((B, F), jnp.float32),
            pltpu.VMEM_SHARED((N + 8, F), jnp.float32),
            pltpu.SemaphoreType.DMA,
            pltpu.SemaphoreType.DMA,
        ],
    )
    def spmm_kernel(src_hbm, dst_hbm, g_hbm, zeros_hbm, out,
                    src_v, dst_v, rows0, rows1, acc, sem0, sem1):
        c = lax.axis_index("c")
        s = lax.axis_index("s")
        row0 = jnp.minimum(s * RPT, N - RPT)
        pltpu.sync_copy(
            zeros_hbm.at[pl.ds(row0, RPT)], acc.at[pl.ds(row0, RPT)]
        )
        plsc.subcore_barrier()

        rows = (rows0, rows1)
        sems = (sem0, sem1)

        def gbody(g, carry):
            base = c * CHH + g * 16
            pltpu.sync_copy(src_hbm.at[s, pl.ds(base, 16)], src_v)
            pltpu.sync_copy(dst_hbm.at[s, pl.ds(base, 16)], dst_v)
            pltpu.async_copy(g_hbm.at[src_v.at[0]], rows0, sem0)
            pltpu.async_copy(g_hbm.at[src_v.at[1]], rows1, sem1)

            def body(jj, c2):
                for b in range(2):
                    j = 2 * jj + b
                    jn = jnp.minimum(j + 2, 15)
                    pltpu.make_async_copy(
                        g_hbm.at[src_v.at[j]], rows[b], sems[b]
                    ).wait()
                    pltpu.sync_copy(rows[b], acc.at[dst_v.at[j]], add=True)
                    pltpu.async_copy(g_hbm.at[src_v.at[jn]], rows[b], sems[b])
                return c2

            lax.fori_loop(0, 8, body, 0)
            for b in range(2):
                pltpu.make_async_copy(
                    g_hbm.at[src_v.at[15]], rows[b], sems[b]
                ).wait()
            return carry

        lax.fori_loop(0, CHH // 16, gbody, 0)
        plsc.subcore_barrier()
        pltpu.sync_copy(
            acc.at[pl.ds(row0, RPT)], out.at[c, pl.ds(row0, RPT)]
        )

    return spmm_kernel


def _make_deg(N, CH):
    F = 128
    CHH = CH // NC
    RPT = _rpt(N)

    @functools.partial(
        pl.kernel,
        out_type=jax.ShapeDtypeStruct((NC, N, F), jnp.float32),
        mesh=_mesh(),
        scratch_types=[
            pltpu.VMEM((16, B), jnp.int32),
            pltpu.VMEM((B, F), jnp.float32),
            pltpu.VMEM_SHARED((N + 8, F), jnp.float32),
        ],
    )
    def deg_kernel(dst_hbm, ones_hbm, zeros_hbm, out, dst_v, ones_v, acc):
        c = lax.axis_index("c")
        s = lax.axis_index("s")
        row0 = jnp.minimum(s * RPT, N - RPT)
        pltpu.sync_copy(ones_hbm, ones_v)
        pltpu.sync_copy(
            zeros_hbm.at[pl.ds(row0, RPT)], acc.at[pl.ds(row0, RPT)]
        )
        plsc.subcore_barrier()

        def gbody(g, carry):
            pltpu.sync_copy(dst_hbm.at[s, pl.ds(c * CHH + g * 16, 16)], dst_v)

            def body(j, carry2):
                pltpu.sync_copy(ones_v, acc.at[dst_v.at[j]], add=True)
                return carry2

            lax.fori_loop(0, 16, body, 0)
            return carry

        lax.fori_loop(0, CHH // 16, gbody, 0)
        plsc.subcore_barrier()
        pltpu.sync_copy(
            acc.at[pl.ds(row0, RPT)], out.at[c, pl.ds(row0, RPT)]
        )

    return deg_kernel


def _dinv_block(d0, d1):
    d = d0[:, 0:1] + d1[:, 0:1]
    return jnp.where(d > 0.0, lax.rsqrt(d), 0.0)


def _pre_body(x_ref, d0_ref, d1_ref, g_ref):
    dinv = _dinv_block(d0_ref[...], d1_ref[...])
    g_ref[...] = x_ref[...] * dinv


def _mid_body(s_ref, d0_ref, d1_ref, tx_ref, g_ref):
    dinv = _dinv_block(d0_ref[...], d1_ref[...])
    sfull = s_ref[0] + s_ref[1]
    tx = -(dinv * sfull)
    tx_ref[...] = tx
    g_ref[...] = dinv * tx


def _comb1_body(x_ref, tx1_ref, s2_ref, d0_ref, d1_ref, w_ref, b_ref, h_ref, g_ref):
    dinv = _dinv_block(d0_ref[...], d1_ref[...])
    s2 = s2_ref[0] + s2_ref[1]
    xb = x_ref[...]
    tx2 = -2.0 * (dinv * s2) - xb
    acc = (
        jnp.dot(xb, w_ref[0], preferred_element_type=jnp.float32)
        + jnp.dot(tx1_ref[...], w_ref[1], preferred_element_type=jnp.float32)
        + jnp.dot(tx2, w_ref[2], preferred_element_type=jnp.float32)
        + b_ref[...]
    )
    h = jnp.maximum(acc, 0.0)
    h_ref[...] = h
    g_ref[...] = dinv * h


def _comb2_body(h_ref, ty1_ref, s4_ref, d0_ref, d1_ref, w_ref, b_ref, o_ref):
    dinv = _dinv_block(d0_ref[...], d1_ref[...])
    s4 = s4_ref[0] + s4_ref[1]
    hb = h_ref[...]
    ty2 = -2.0 * (dinv * s4) - hb
    logits = (
        jnp.dot(hb, w_ref[0], preferred_element_type=jnp.float32)
        + jnp.dot(ty1_ref[...], w_ref[1], preferred_element_type=jnp.float32)
        + jnp.dot(ty2, w_ref[2], preferred_element_type=jnp.float32)
        + b_ref[...]
    )
    m = jnp.max(logits, axis=1, keepdims=True)
    lse = jnp.log(jnp.sum(jnp.exp(logits - m), axis=1, keepdims=True)) + m
    o_ref[...] = logits - lse


def _row_spec(w):
    return pl.BlockSpec((R, w), lambda i: (i, 0))


def _pair_spec(w):
    return pl.BlockSpec((2, R, w), lambda i: (0, i, 0))


def kernel(x, edge_index, W1, b1, W2, b2):
    N, F = x.shape
    E = edge_index.shape[1]
    H = W1.shape[2]
    C = W2.shape[2]
    NB = N // R

    CH = -(-E // (NS * B * 16)) * 16
    Epad = NS * CH * B
    src = edge_index[0]
    dst = edge_index[1]
    pad = Epad - E
    srcp = jnp.concatenate([src, jnp.zeros((pad,), jnp.int32)]).reshape(NS, CH, B)
    dstp = jnp.concatenate([dst, jnp.full((pad,), N, jnp.int32)]).reshape(NS, CH, B)
    zerosF = jnp.zeros((N, F), jnp.float32)
    onesB = jnp.ones((B, F), jnp.float32)

    spmm = _make_spmm(N, CH, F)
    degp = _make_deg(N, CH)(dstp, onesB, zerosF)
    deg0 = degp[0, :, :8]
    deg1 = degp[1, :, :8]

    dspec = _row_spec(8)

    pre = pl.pallas_call(
        _pre_body,
        grid=(NB,),
        in_specs=[_row_spec(F), dspec, dspec],
        out_specs=_row_spec(F),
        out_shape=jax.ShapeDtypeStruct((N, F), jnp.float32),
    )
    mid = pl.pallas_call(
        _mid_body,
        grid=(NB,),
        in_specs=[_pair_spec(F), dspec, dspec],
        out_specs=[_row_spec(F), _row_spec(F)],
        out_shape=[
            jax.ShapeDtypeStruct((N, F), jnp.float32),
            jax.ShapeDtypeStruct((N, F), jnp.float32),
        ],
    )
    comb1 = pl.pallas_call(
        _comb1_body,
        grid=(NB,),
        in_specs=[
            _row_spec(F),
            _row_spec(F),
            _pair_spec(F),
            dspec,
            dspec,
            pl.BlockSpec((3, F, H), lambda i: (0, 0, 0)),
            pl.BlockSpec((1, H), lambda i: (0, 0)),
        ],
        out_specs=[_row_spec(H), _row_spec(H)],
        out_shape=[
            jax.ShapeDtypeStruct((N, H), jnp.float32),
            jax.ShapeDtypeStruct((N, H), jnp.float32),
        ],
    )
    comb2 = pl.pallas_call(
        _comb2_body,
        grid=(NB,),
        in_specs=[
            _row_spec(H),
            _row_spec(H),
            _pair_spec(H),
            dspec,
            dspec,
            pl.BlockSpec((3, H, C), lambda i: (0, 0, 0)),
            pl.BlockSpec((1, C), lambda i: (0, 0)),
        ],
        out_specs=_row_spec(C),
        out_shape=jax.ShapeDtypeStruct((N, C), jnp.float32),
    )

    g0 = pre(x, deg0, deg1)
    s1 = spmm(srcp, dstp, g0, zerosF)
    tx1, g1 = mid(s1, deg0, deg1)
    s2 = spmm(srcp, dstp, g1, zerosF)
    h, g2 = comb1(x, tx1, s2, deg0, deg1, W1, b1.reshape(1, H))
    s3 = spmm(srcp, dstp, g2, zerosF)
    ty1, g3 = mid(s3, deg0, deg1)
    s4 = spmm(srcp, dstp, g3, zerosF)
    out = comb2(h, ty1, s4, deg0, deg1, W2, b2.reshape(1, C))
    return out
